# baseline (speedup 1.0000x reference)
import math

import jax
import jax.numpy as jnp
from jax import lax
from jax.experimental import pallas as pl
from jax.experimental.pallas import tpu as pltpu

_F = 26
_C = 1000
_D = 16
_K = _F * _C

_BK = 2000
_NK = _K // _BK


def _mm_body(z_ref, w_ref, o_ref):
    bm13, _ = z_ref.shape
    p = jnp.dot(z_ref[:], w_ref[:], preferred_element_type=jnp.float32)
    r = lax.broadcasted_iota(jnp.int32, p.shape, 0)
    c = lax.broadcasted_iota(jnp.int32, p.shape, 1)
    keep = (c // _D) == (r % _NK)
    pm = jnp.where(keep, p, 0.0)
    o_ref[:] = pm.reshape(bm13 // _NK, _NK, _NK * _D).sum(axis=(1,)).reshape(
        bm13 // _NK, _NK, _D
    ).sum(axis=1)


def kernel(z, codebook):
    batch_shape = z.shape[:-1]
    m = math.prod(batch_shape)
    zf = z.reshape(m * _NK, _BK)
    w_big = codebook.reshape(_NK, _BK, _D).transpose(1, 0, 2).reshape(
        _BK, _NK * _D
    )

    bm = 128
    nm = m // bm

    out = pl.pallas_call(
        _mm_body,
        grid=(nm,),
        in_specs=[
            pl.BlockSpec((bm * _NK, _BK), lambda i: (i, 0)),
            pl.BlockSpec((_BK, _NK * _D), lambda i: (0, 0)),
        ],
        out_specs=pl.BlockSpec((bm, _D), lambda i: (i, 0)),
        out_shape=jax.ShapeDtypeStruct((m, _D), jnp.float32),
        compiler_params=pltpu.CompilerParams(
            dimension_semantics=("parallel",)
        ),
    )(zf, w_big)
    return out.reshape(*batch_shape, _D)


# speedup vs baseline: 1.6555x; 1.6555x over previous
import math

import jax
import jax.numpy as jnp
from jax.experimental import pallas as pl
from jax.experimental.pallas import tpu as pltpu

_F = 26
_C = 1000
_D = 16
_K = _F * _C

_BM = 64
_NBUF = 4


def _mm_body(z_hbm, w_ref, o_ref, buf, sems):
    i = pl.program_id(0)
    nm = pl.num_programs(0)

    def copy(chunk, slot):
        return pltpu.make_async_copy(
            z_hbm.at[pl.ds(chunk * _BM, _BM), :],
            buf.at[slot],
            sems.at[slot],
        )

    @pl.when(i == 0)
    def _warmup():
        for s in range(_NBUF - 1):
            copy(s, s).start()

    nxt = i + _NBUF - 1

    @pl.when(nxt < nm)
    def _prefetch():
        copy(nxt, jax.lax.rem(nxt, _NBUF)).start()

    slot = jax.lax.rem(i, _NBUF)
    copy(i, slot).wait()
    o_ref[:] = jnp.dot(buf[slot], w_ref[:], preferred_element_type=jnp.float32)


def kernel(z, codebook):
    batch_shape = z.shape[:-1]
    m = math.prod(batch_shape)
    z2 = z.reshape(m, _K)
    w = codebook.reshape(_K, _D)

    out = pl.pallas_call(
        _mm_body,
        grid=(m // _BM,),
        in_specs=[
            pl.BlockSpec(memory_space=pltpu.MemorySpace.HBM),
            pl.BlockSpec((_K, _D), lambda i: (0, 0)),
        ],
        out_specs=pl.BlockSpec((_BM, _D), lambda i: (i, 0)),
        out_shape=jax.ShapeDtypeStruct((m, _D), jnp.float32),
        scratch_shapes=[
            pltpu.VMEM((_NBUF, _BM, _K), jnp.float32),
            pltpu.SemaphoreType.DMA((_NBUF,)),
        ],
    )(z2, w)
    return out.reshape(*batch_shape, _D)
